# baseline (speedup 1.0000x reference)
import functools

import jax
import jax.numpy as jnp
from jax import lax
from jax.experimental import pallas as pl
from jax.experimental.pallas import tpu as pltpu
from jax.experimental.pallas import tpu_sc as plsc

_ALT_VALS = (150, 200, 250, 300)
_EMBED_D = 32
_BATCH = 16384
_LANES = 16

_INFO = plsc.get_sparse_core_info()
_NC = _INFO.num_cores
_NS = _INFO.num_subcores
_NW = _NC * _NS
_BPW = _BATCH // _NW
_WPW = _BPW * _EMBED_D
_CHUNK = 128
_NCHUNK = _BPW // _CHUNK
_GPC = _CHUNK // _LANES


def _sc_body(alt_hbm, table_hbm, out_hbm, alt_v, table_v, rows_v, osem):
    wid = lax.axis_index("s") * _NC + lax.axis_index("c")
    base = wid * _BPW

    pltpu.sync_copy(table_hbm, table_v)
    pltpu.sync_copy(alt_hbm.at[pl.ds(base, _BPW)], alt_v)

    lane_pos = lax.iota(jnp.int32, _LANES) * _EMBED_D

    writebacks = []
    for j in range(_NCHUNK):
        for i in range(_GPC):
            g = j * _GPC + i
            a = alt_v[pl.ds(g * _LANES, _LANES)]
            row = jnp.where(a == _ALT_VALS[1], jnp.int32(1 * _EMBED_D),
                            jnp.int32(0))
            row = jnp.where(a == _ALT_VALS[2], jnp.int32(2 * _EMBED_D), row)
            row = jnp.where(a == _ALT_VALS[3], jnp.int32(3 * _EMBED_D), row)
            pos = lane_pos + g * (_LANES * _EMBED_D)
            for d in range(_EMBED_D):
                vals = plsc.load_gather(table_v, [row + d])
                plsc.store_scatter(rows_v, [pos + d], vals)
        writebacks.append(
            pltpu.async_copy(
                rows_v.at[pl.ds(j * _CHUNK * _EMBED_D, _CHUNK * _EMBED_D)],
                out_hbm.at[pl.ds(base * _EMBED_D + j * _CHUNK * _EMBED_D,
                                 _CHUNK * _EMBED_D)],
                osem,
            )
        )
    for w in writebacks:
        w.wait()


_sc_lookup = functools.partial(
    pl.kernel,
    mesh=plsc.VectorSubcoreMesh(core_axis_name="c", subcore_axis_name="s"),
    compiler_params=pltpu.CompilerParams(needs_layout_passes=False),
    out_type=jax.ShapeDtypeStruct((_BATCH * _EMBED_D,), jnp.float32),
    scratch_types=[
        pltpu.VMEM((_BPW,), jnp.int32),
        pltpu.VMEM(((len(_ALT_VALS) + 1) * _EMBED_D,), jnp.float32),
        pltpu.VMEM((_WPW,), jnp.float32),
        pltpu.SemaphoreType.DMA,
    ],
)(_sc_body)


def kernel(altitudes, embeddings):
    flat = _sc_lookup(altitudes, embeddings.reshape(-1))
    return flat.reshape(_BATCH, _EMBED_D)


# speedup vs baseline: 1.3789x; 1.3789x over previous
import functools

import jax
import jax.numpy as jnp
from jax import lax
from jax.experimental import pallas as pl
from jax.experimental.pallas import tpu as pltpu
from jax.experimental.pallas import tpu_sc as plsc

_ALT_VALS = (150, 200, 250, 300)
_EMBED_D = 32
_BATCH = 16384
_LANES = 16

_INFO = plsc.get_sparse_core_info()
_NC = _INFO.num_cores
_NS = _INFO.num_subcores
_NW = _NC * _NS
_BPW = _BATCH // _NW
_CHUNK = 128
_NCHUNK = _BPW // _CHUNK
_GPC = _CHUNK // _LANES


def _sc_body(alt_hbm, table_hbm, out_hbm, alt_v, idx_v, table_sh, rows_v,
             gsems, osem):
    wid = lax.axis_index("s") * _NC + lax.axis_index("c")
    base = wid * _BPW

    @pl.when(lax.axis_index("s") == 0)
    def _():
        pltpu.sync_copy(table_hbm, table_sh)

    pltpu.sync_copy(alt_hbm.at[pl.ds(base, _BPW)], alt_v)
    plsc.subcore_barrier()

    def compute_and_fire_gather(j):
        for i in range(_GPC):
            off = j * _CHUNK + i * _LANES
            a = alt_v[pl.ds(off, _LANES)]
            idx = jnp.where(a == _ALT_VALS[1], jnp.int32(1), jnp.int32(0))
            idx = jnp.where(a == _ALT_VALS[2], jnp.int32(2), idx)
            idx = jnp.where(a == _ALT_VALS[3], jnp.int32(3), idx)
            idx_v[pl.ds(off, _LANES)] = idx
        return pltpu.async_copy(
            table_sh.at[idx_v.at[pl.ds(j * _CHUNK, _CHUNK)]],
            rows_v.at[pl.ds(j * _CHUNK, _CHUNK)],
            gsems[j],
        )

    def fire_writeback(j):
        return pltpu.async_copy(
            rows_v.at[pl.ds(j * _CHUNK, _CHUNK)],
            out_hbm.at[pl.ds(base + j * _CHUNK, _CHUNK)],
            osem,
        )

    gathers = [compute_and_fire_gather(0), compute_and_fire_gather(1)]
    writebacks = []
    for j in range(_NCHUNK):
        gathers[j].wait()
        writebacks.append(fire_writeback(j))
        if j + 2 < _NCHUNK:
            gathers.append(compute_and_fire_gather(j + 2))
    for w in writebacks:
        w.wait()


_sc_lookup = functools.partial(
    pl.kernel,
    mesh=plsc.VectorSubcoreMesh(core_axis_name="c", subcore_axis_name="s"),
    compiler_params=pltpu.CompilerParams(
        use_tc_tiling_on_sc=False,
        skip_device_barrier=True,
        disable_bounds_checks=True,
        disable_semaphore_checks=True,
    ),
    out_type=jax.ShapeDtypeStruct((_BATCH, _EMBED_D), jnp.float32),
    scratch_types=[
        pltpu.VMEM((_BPW,), jnp.int32),
        pltpu.VMEM((_BPW,), jnp.int32),
        pltpu.VMEM_SHARED((len(_ALT_VALS) + 1, _EMBED_D), jnp.float32),
        pltpu.VMEM((_BPW, _EMBED_D), jnp.float32),
        [pltpu.SemaphoreType.DMA] * _NCHUNK,
        pltpu.SemaphoreType.DMA,
    ],
)(_sc_body)


def kernel(altitudes, embeddings):
    return _sc_lookup(altitudes, embeddings)


# speedup vs baseline: 1.5619x; 1.1327x over previous
import functools

import jax
import jax.numpy as jnp
from jax import lax
from jax.experimental import pallas as pl
from jax.experimental.pallas import tpu as pltpu
from jax.experimental.pallas import tpu_sc as plsc

_ALT_VALS = (150, 200, 250, 300)
_EMBED_D = 32
_BATCH = 16384
_LANES = 16

_INFO = plsc.get_sparse_core_info()
_NC = _INFO.num_cores
_NS = _INFO.num_subcores
_NW = _NC * _NS
_BPW = _BATCH // _NW
_CHUNK = 128
_NCHUNK = _BPW // _CHUNK


def _sc_body(alt_hbm, table_hbm, out_hbm, alt_v, idx_v, table_v, rows_v, sem):
    wid = lax.axis_index("s") * _NC + lax.axis_index("c")
    base = wid * _BPW

    @pl.when(lax.axis_index("s") == 0)
    def _():
        pltpu.sync_copy(table_hbm, table_v)

    pltpu.sync_copy(alt_hbm.at[pl.ds(base, _BPW)], alt_v)
    plsc.subcore_barrier()

    for i in range(_BPW // _LANES):
        a = alt_v[pl.ds(i * _LANES, _LANES)]
        idx = jnp.where(a == _ALT_VALS[1], jnp.int32(1), jnp.int32(0))
        idx = jnp.where(a == _ALT_VALS[2], jnp.int32(2), idx)
        idx = jnp.where(a == _ALT_VALS[3], jnp.int32(3), idx)
        idx_v[pl.ds(i * _LANES, _LANES)] = idx

    copies = []
    for j in range(_NCHUNK):
        copies.append(
            pltpu.async_copy(
                table_v.at[idx_v.at[pl.ds(j * _CHUNK, _CHUNK)]],
                rows_v.at[pl.ds(j * _CHUNK, _CHUNK)],
                sem,
            )
        )
    for c in copies:
        c.wait()

    pltpu.sync_copy(rows_v, out_hbm.at[pl.ds(base, _BPW)])


_sc_lookup = functools.partial(
    pl.kernel,
    mesh=plsc.VectorSubcoreMesh(core_axis_name="c", subcore_axis_name="s"),
    compiler_params=pltpu.CompilerParams(
        skip_device_barrier=True,
        disable_bounds_checks=True,
        disable_semaphore_checks=True,
    ),
    out_type=jax.ShapeDtypeStruct((_BATCH, _EMBED_D), jnp.float32),
    scratch_types=[
        pltpu.VMEM((_BPW,), jnp.int32),
        pltpu.VMEM((_BPW,), jnp.int32),
        pltpu.VMEM_SHARED((len(_ALT_VALS) + 1, _EMBED_D), jnp.float32),
        pltpu.VMEM((_BPW, _EMBED_D), jnp.float32),
        pltpu.SemaphoreType.DMA,
    ],
)(_sc_body)


def kernel(altitudes, embeddings):
    return _sc_lookup(altitudes, embeddings)


# speedup vs baseline: 1.5664x; 1.0029x over previous
import functools

import jax
import jax.numpy as jnp
from jax import lax
from jax.experimental import pallas as pl
from jax.experimental.pallas import tpu as pltpu
from jax.experimental.pallas import tpu_sc as plsc

_ALT_VALS = (150, 200, 250, 300)
_EMBED_D = 32
_BATCH = 16384
_LANES = 16

_INFO = plsc.get_sparse_core_info()
_NC = _INFO.num_cores
_NS = _INFO.num_subcores
_NW = _NC * _NS
_BPW = _BATCH // _NW
_CHUNK = 128
_NCHUNK = _BPW // _CHUNK


def _sc_body(alt_hbm, table_hbm, out_hbm, alt_v, idx_v, table_v, rows_v,
             gsems, osem):
    wid = lax.axis_index("s") * _NC + lax.axis_index("c")
    base = wid * _BPW

    @pl.when(lax.axis_index("s") == 0)
    def _():
        pltpu.sync_copy(table_hbm, table_v)

    pltpu.sync_copy(alt_hbm.at[pl.ds(base, _BPW)], alt_v)
    plsc.subcore_barrier()

    def compute_and_fire_gather(j):
        for i in range(_CHUNK // _LANES):
            off = j * _CHUNK + i * _LANES
            a = alt_v[pl.ds(off, _LANES)]
            idx = jnp.where(a == _ALT_VALS[1], jnp.int32(1), jnp.int32(0))
            idx = jnp.where(a == _ALT_VALS[2], jnp.int32(2), idx)
            idx = jnp.where(a == _ALT_VALS[3], jnp.int32(3), idx)
            idx_v[pl.ds(off, _LANES)] = idx
        return pltpu.async_copy(
            table_v.at[idx_v.at[pl.ds(j * _CHUNK, _CHUNK)]],
            rows_v.at[pl.ds(j * _CHUNK, _CHUNK)],
            gsems[j],
        )

    def fire_writeback(j):
        return pltpu.async_copy(
            rows_v.at[pl.ds(j * _CHUNK, _CHUNK)],
            out_hbm.at[pl.ds(base + j * _CHUNK, _CHUNK)],
            osem,
        )

    gathers = [compute_and_fire_gather(0), compute_and_fire_gather(1)]
    writebacks = []
    for j in range(_NCHUNK):
        gathers[j].wait()
        writebacks.append(fire_writeback(j))
        if j + 2 < _NCHUNK:
            gathers.append(compute_and_fire_gather(j + 2))
    for w in writebacks:
        w.wait()


_sc_lookup = functools.partial(
    pl.kernel,
    mesh=plsc.VectorSubcoreMesh(core_axis_name="c", subcore_axis_name="s"),
    compiler_params=pltpu.CompilerParams(
        skip_device_barrier=True,
        disable_bounds_checks=True,
        disable_semaphore_checks=True,
    ),
    out_type=jax.ShapeDtypeStruct((_BATCH, _EMBED_D), jnp.float32),
    scratch_types=[
        pltpu.VMEM((_BPW,), jnp.int32),
        pltpu.VMEM((_BPW,), jnp.int32),
        pltpu.VMEM_SHARED((len(_ALT_VALS) + 1, _EMBED_D), jnp.float32),
        pltpu.VMEM((_BPW, _EMBED_D), jnp.float32),
        [pltpu.SemaphoreType.DMA] * _NCHUNK,
        pltpu.SemaphoreType.DMA,
    ],
)(_sc_body)


def kernel(altitudes, embeddings):
    return _sc_lookup(altitudes, embeddings)
